# baseline (speedup 1.0000x reference)
import functools

import jax
import jax.numpy as jnp
from jax import lax
from jax.experimental import pallas as pl
from jax.experimental.pallas import tpu as pltpu
from jax.experimental.pallas import tpu_sc as plsc

N = 100_000
G = 2048
H = 128
VP = 128
CHUNK = 4096
NW = 32
GPW = G // NW
BBUF = 48


def _bounds_body(b_ref, out_ref):
    b = b_ref[...]
    out_ref[0] = jnp.int32(0)
    for t in range(1, NW):
        out_ref[t] = jnp.sum((b < t * GPW).astype(jnp.int32))
    out_ref[NW] = jnp.int32(N)
    for t in range(NW + 1, BBUF):
        out_ref[t] = jnp.int32(0)


def _bounds_call(b2d):
    return pl.pallas_call(
        _bounds_body,
        out_shape=jax.ShapeDtypeStruct((BBUF,), jnp.int32),
        in_specs=[pl.BlockSpec(memory_space=pltpu.VMEM)],
        out_specs=pl.BlockSpec(memory_space=pltpu.SMEM),
    )(b2d)


_sc_mesh = plsc.VectorSubcoreMesh(core_axis_name="c", subcore_axis_name="s")


@functools.partial(
    pl.kernel,
    mesh=_sc_mesh,
    out_type=jax.ShapeDtypeStruct((G, VP), jnp.float32),
    scratch_types=[
        pltpu.VMEM((CHUNK,), jnp.int32),
        pltpu.VMEM((CHUNK,), jnp.int32),
        pltpu.VMEM((CHUNK,), jnp.int32),
        pltpu.VMEM((CHUNK,), jnp.int32),
        pltpu.VMEM((GPW, VP), jnp.float32),
        pltpu.VMEM((BBUF,), jnp.int32),
        pltpu.SemaphoreType.DMA,
        pltpu.SemaphoreType.DMA,
        pltpu.SemaphoreType.DMA,
        pltpu.SemaphoreType.DMA,
    ],
    compiler_params=pltpu.CompilerParams(needs_layout_passes=False),
)
def _sc_hist(z_hbm, b_hbm, bounds_hbm, out_hbm, zbufa, zbufb, bbufa, bbufb,
             hist, bnd, semz0, semz1, semb0, semb1):
    wid = lax.axis_index("s") * 2 + lax.axis_index("c")
    pltpu.sync_copy(bounds_hbm, bnd)

    lo = bnd[pl.ds(wid, 16)][0]
    hi = bnd[pl.ds(wid + 1, 16)][0]

    zeros16 = jnp.zeros((16,), jnp.float32)
    ones16 = jnp.ones((16,), jnp.float32)
    lanes = jnp.arange(16, dtype=jnp.int32)

    g_base = wid * GPW
    c0 = lo // CHUNK
    c1 = (hi + CHUNK - 1) // CHUNK
    nch = c1 - c0

    def chunk_base(c):
        return jnp.minimum(c * CHUNK, N - CHUNK)

    def issue(c, zdst, bdst, sz, sb):
        base = chunk_base(c)
        cz = pltpu.async_copy(z_hbm.at[pl.ds(base, CHUNK)], zdst, sz)
        cb = pltpu.async_copy(b_hbm.at[pl.ds(base, CHUNK)], bdst, sb)
        return cz, cb

    @pl.when(nch >= 1)
    def _():
        issue(c0, zbufa, bbufa, semz0, semb0)

    @pl.when(nch >= 2)
    def _():
        issue(c0 + 1, zbufb, bbufb, semz1, semb1)

    def zero_body(r, carry):
        for j in range(VP // 16):
            hist[r, pl.ds(j * 16, 16)] = zeros16
        return carry

    lax.fori_loop(0, GPW, zero_body, 0, unroll=2)

    def process(zb, bb, c):
        start = c * CHUNK
        base = chunk_base(c)
        lo_i4 = (jnp.maximum(lo, start) - base) // 64
        hi_i4 = (jnp.minimum(hi, base + CHUNK) - base + 63) // 64

        def vec_body4(i4, inner):
            for u in range(4):
                i = i4 * 4 + u
                zv = zb[pl.ds(i * 16, 16)]
                bv = bb[pl.ds(i * 16, 16)]
                rel = bv - g_base
                msk = ((rel >= 0) & (rel < GPW)
                       & (base + i * 16 + lanes >= start))
                rel = jnp.where(msk, rel, 0)
                plsc.addupdate_scatter(hist, [rel, zv], ones16, mask=msk)
            return inner

        lax.fori_loop(lo_i4, hi_i4, vec_body4, 0)

    def wait(zdst, bdst, sz, sb):
        pltpu.make_async_copy(z_hbm.at[pl.ds(0, CHUNK)], zdst, sz).wait()
        pltpu.make_async_copy(b_hbm.at[pl.ds(0, CHUNK)], bdst, sb).wait()

    @pl.when(nch >= 1)
    def _():
        wait(zbufa, bbufa, semz0, semb0)
        process(zbufa, bbufa, c0)

    @pl.when(nch >= 2)
    def _():
        wait(zbufb, bbufb, semz1, semb1)
        process(zbufb, bbufb, c0 + 1)

    def extra_body(c, carry):
        cz, cb = issue(c, zbufa, bbufa, semz0, semb0)
        cz.wait()
        cb.wait()
        process(zbufa, bbufa, c)
        return carry

    lax.fori_loop(c0 + 2, c1, extra_body, 0)

    pltpu.sync_copy(hist, out_hbm.at[pl.ds(g_base, GPW)])


def _head_body(hist_ref, emb_ref, w1_ref, b1_ref, w2_ref, b2_ref, out_ref):
    hist = hist_ref[...]
    counts = jnp.sum(hist, axis=1, keepdims=True)
    denom = jnp.maximum(counts, 1.0)
    V = emb_ref.shape[0]
    sums = lax.dot_general(hist[:, :V], emb_ref[...],
                           (((1,), (0,)), ((), ())),
                           preferred_element_type=jnp.float32)
    pooled = sums / denom
    h = jnp.maximum(
        lax.dot_general(pooled, w1_ref[...], (((1,), (1,)), ((), ())),
                        preferred_element_type=jnp.float32) + b1_ref[...],
        0.0)
    out = jnp.sum(h * w2_ref[...], axis=1) + b2_ref[0]
    out_ref[...] = out


def _head_call(hist, emb, w1, b1, w2, b2):
    return pl.pallas_call(
        _head_body,
        out_shape=jax.ShapeDtypeStruct((G,), jnp.float32),
        in_specs=[
            pl.BlockSpec(memory_space=pltpu.VMEM),
            pl.BlockSpec(memory_space=pltpu.VMEM),
            pl.BlockSpec(memory_space=pltpu.VMEM),
            pl.BlockSpec(memory_space=pltpu.VMEM),
            pl.BlockSpec(memory_space=pltpu.VMEM),
            pl.BlockSpec(memory_space=pltpu.SMEM),
        ],
        out_specs=pl.BlockSpec(memory_space=pltpu.VMEM),
    )(hist, emb, w1, b1, w2, b2)


def kernel(z, batch_ids, emb, W1, b1, W2, b2):
    z = z.astype(jnp.int32)
    b = batch_ids.astype(jnp.int32)

    bounds = _bounds_call(b.reshape(100, 1000))
    hist = _sc_hist(z, b, bounds)

    out = _head_call(
        hist,
        emb.astype(jnp.float32),
        W1.astype(jnp.float32),
        b1.astype(jnp.float32).reshape(1, H),
        W2.astype(jnp.float32),
        b2.astype(jnp.float32),
    )
    return out.reshape(G, 1)
